# baseline (speedup 1.0000x reference)
import jax
import jax.numpy as jnp
from jax.experimental import pallas as pl
from jax.experimental.pallas import tpu as pltpu

_N = 10000
_F = 128
_HEADS = 8
_HID = 32
_HH = _HEADS * _HID
_G = 64
_C = 10
_BN = 1000


def _proj1_body(x_ref, w_ref, as_ref, ad_ref, h_ref, asrc_ref, adst_ref):
    h = jnp.dot(x_ref[...], w_ref[...], preferred_element_type=jnp.float32)
    h_ref[...] = h
    hh = h.reshape(_BN, _HEADS, _HID)
    asrc_ref[...] = jnp.sum(hh * as_ref[...][None], axis=-1)
    adst_ref[...] = jnp.sum(hh * ad_ref[...][None], axis=-1)


def _proj2_body(num_ref, den_ref, b_ref, w_ref, as_ref, ad_ref,
                h_ref, asrc_ref, adst_ref):
    agg = num_ref[...].reshape(_BN, _HEADS, _HID)
    den = den_ref[...][:, :, None]
    hin = agg / (den + 1e-16)
    hin = hin.reshape(_BN, _HH) + b_ref[...]
    hin = jnp.maximum(hin, 0.0)
    h = jnp.dot(hin, w_ref[...], preferred_element_type=jnp.float32)
    h_ref[...] = h
    hh = h.reshape(_BN, _HEADS, _HID)
    asrc_ref[...] = jnp.sum(hh * as_ref[...][None], axis=-1)
    adst_ref[...] = jnp.sum(hh * ad_ref[...][None], axis=-1)


def _pool_body(num_ref, den_ref, b_ref, batch_ref, wfc_ref, bfc_ref,
               out_ref, sums_ref, cnt_ref):
    i = pl.program_id(0)
    nb = pl.num_programs(0)

    @pl.when(i == 0)
    def _():
        sums_ref[...] = jnp.zeros_like(sums_ref)
        cnt_ref[...] = jnp.zeros_like(cnt_ref)

    agg = num_ref[...].reshape(_BN, _HEADS, _HID)
    den = den_ref[...][:, :, None]
    h = agg / (den + 1e-16)
    h = h.reshape(_BN, _HH) + b_ref[...]
    h = jnp.maximum(h, 0.0)

    seg = batch_ref[...]
    gids = jax.lax.broadcasted_iota(jnp.int32, (_BN, _G), 1)
    onehot = (gids == seg).astype(jnp.float32)
    sums_ref[...] += jax.lax.dot_general(
        onehot, h, (((0,), (0,)), ((), ())),
        preferred_element_type=jnp.float32)
    cnt_ref[...] += jnp.sum(onehot, axis=0).reshape(_G, 1)

    @pl.when(i == nb - 1)
    def _():
        pooled = sums_ref[...] / jnp.maximum(cnt_ref[...], 1.0)
        out_ref[...] = jnp.dot(
            pooled, wfc_ref[...],
            preferred_element_type=jnp.float32) + bfc_ref[...]


def _full(shape):
    return pl.BlockSpec(shape, lambda i: tuple(0 for _ in shape))


def _rows(shape):
    return pl.BlockSpec(shape, lambda i: (i,) + tuple(0 for _ in shape[1:]))


def _proj1(x, W, a_s, a_d):
    grid = _N // _BN
    return pl.pallas_call(
        _proj1_body,
        grid=(grid,),
        in_specs=[_rows((_BN, _F)), _full((_F, _HH)),
                  _full((_HEADS, _HID)), _full((_HEADS, _HID))],
        out_specs=[_rows((_BN, _HH)), _rows((_BN, _HEADS)), _rows((_BN, _HEADS))],
        out_shape=[jax.ShapeDtypeStruct((_N, _HH), jnp.float32),
                   jax.ShapeDtypeStruct((_N, _HEADS), jnp.float32),
                   jax.ShapeDtypeStruct((_N, _HEADS), jnp.float32)],
    )(x, W, a_s, a_d)


def _proj2(numer, denom, b, W, a_s, a_d):
    grid = _N // _BN
    return pl.pallas_call(
        _proj2_body,
        grid=(grid,),
        in_specs=[_rows((_BN, _HH)), _rows((_BN, _HEADS)), _full((1, _HH)),
                  _full((_HH, _HH)), _full((_HEADS, _HID)), _full((_HEADS, _HID))],
        out_specs=[_rows((_BN, _HH)), _rows((_BN, _HEADS)), _rows((_BN, _HEADS))],
        out_shape=[jax.ShapeDtypeStruct((_N, _HH), jnp.float32),
                   jax.ShapeDtypeStruct((_N, _HEADS), jnp.float32),
                   jax.ShapeDtypeStruct((_N, _HEADS), jnp.float32)],
    )(numer, denom, b.reshape(1, _HH), W, a_s, a_d)


def _pool_fc(numer, denom, b, batch, Wfc, bfc):
    grid = _N // _BN
    return pl.pallas_call(
        _pool_body,
        grid=(grid,),
        in_specs=[_rows((_BN, _HH)), _rows((_BN, _HEADS)), _full((1, _HH)),
                  _rows((_BN, 1)), _full((_HH, _C)), _full((1, _C))],
        out_specs=_full((_G, _C)),
        out_shape=jax.ShapeDtypeStruct((_G, _C), jnp.float32),
        scratch_shapes=[pltpu.VMEM((_G, _HH), jnp.float32),
                        pltpu.VMEM((_G, 1), jnp.float32)],
    )(numer, denom, b.reshape(1, _HH), batch.reshape(_N, 1).astype(jnp.int32),
      Wfc, bfc.reshape(1, _C))


def _edge_aggregate(h, asrc, adst, src, dst):
    al = asrc[src] + adst[dst]
    al = jnp.where(al > 0, al, 0.2 * al)
    e = jnp.exp(al)
    msg = (h[src].reshape(-1, _HEADS, _HID) * e[:, :, None]).reshape(-1, _HH)
    both = jax.ops.segment_sum(jnp.concatenate([msg, e], axis=1), dst,
                               num_segments=_N)
    return both[:, :_HH], both[:, _HH:]


def kernel(x, edge_index, batch, W1, a_s1, a_d1, b1, W2, a_s2, a_d2, b2, Wfc, bfc):
    src = edge_index[0]
    dst = edge_index[1]
    h1, asrc1, adst1 = _proj1(x, W1, a_s1, a_d1)
    num1, den1 = _edge_aggregate(h1, asrc1, adst1, src, dst)
    h2, asrc2, adst2 = _proj2(num1, den1, b1, W2, a_s2, a_d2)
    num2, den2 = _edge_aggregate(h2, asrc2, adst2, src, dst)
    return _pool_fc(num2, den2, b2, batch, Wfc, bfc)


# speedup vs baseline: 1.0941x; 1.0941x over previous
import jax
import jax.numpy as jnp
from jax.experimental import pallas as pl
from jax.experimental.pallas import tpu as pltpu

_N = 10000
_F = 128
_HEADS = 8
_HID = 32
_HH = _HEADS * _HID
_G = 64
_C = 10
_BN = 1000


def _proj1_body(x_ref, w_ref, as_ref, ad_ref, h_ref, asrc_ref, adst_ref):
    h = jnp.dot(x_ref[...], w_ref[...], preferred_element_type=jnp.float32)
    h_ref[...] = h
    hh = h.reshape(_BN, _HEADS, _HID)
    asrc_ref[...] = jnp.sum(hh * as_ref[...][None], axis=-1)
    adst_ref[...] = jnp.sum(hh * ad_ref[...][None], axis=-1)


def _proj2_body(num_ref, den_ref, b_ref, w_ref, as_ref, ad_ref,
                h_ref, asrc_ref, adst_ref):
    agg = num_ref[...].reshape(_BN, _HEADS, _HID)
    den = den_ref[...][:, :, None]
    hin = agg / (den + 1e-16)
    hin = hin.reshape(_BN, _HH) + b_ref[...]
    hin = jnp.maximum(hin, 0.0)
    h = jnp.dot(hin, w_ref[...], preferred_element_type=jnp.float32)
    h_ref[...] = h
    hh = h.reshape(_BN, _HEADS, _HID)
    asrc_ref[...] = jnp.sum(hh * as_ref[...][None], axis=-1)
    adst_ref[...] = jnp.sum(hh * ad_ref[...][None], axis=-1)


def _pool_body(num_ref, den_ref, b_ref, batch_ref, wfc_ref, bfc_ref,
               out_ref, sums_ref, cnt_ref):
    i = pl.program_id(0)
    nb = pl.num_programs(0)

    @pl.when(i == 0)
    def _():
        sums_ref[...] = jnp.zeros_like(sums_ref)
        cnt_ref[...] = jnp.zeros_like(cnt_ref)

    agg = num_ref[...].reshape(_BN, _HEADS, _HID)
    den = den_ref[...][:, :, None]
    h = agg / (den + 1e-16)
    h = h.reshape(_BN, _HH) + b_ref[...]
    h = jnp.maximum(h, 0.0)

    seg = batch_ref[...]
    gids = jax.lax.broadcasted_iota(jnp.int32, (_BN, _G), 1)
    onehot = (gids == seg).astype(jnp.float32)
    sums_ref[...] += jax.lax.dot_general(
        onehot, h, (((0,), (0,)), ((), ())),
        preferred_element_type=jnp.float32)
    cnt_ref[...] += jnp.sum(onehot, axis=0).reshape(_G, 1)

    @pl.when(i == nb - 1)
    def _():
        pooled = sums_ref[...] / jnp.maximum(cnt_ref[...], 1.0)
        out_ref[...] = jnp.dot(
            pooled, wfc_ref[...],
            preferred_element_type=jnp.float32) + bfc_ref[...]


def _full(shape):
    return pl.BlockSpec(shape, lambda i: tuple(0 for _ in shape))


def _rows(shape):
    return pl.BlockSpec(shape, lambda i: (i,) + tuple(0 for _ in shape[1:]))


def _proj1(x, W, a_s, a_d):
    grid = _N // _BN
    return pl.pallas_call(
        _proj1_body,
        grid=(grid,),
        in_specs=[_rows((_BN, _F)), _full((_F, _HH)),
                  _full((_HEADS, _HID)), _full((_HEADS, _HID))],
        out_specs=[_rows((_BN, _HH)), _rows((_BN, _HEADS)), _rows((_BN, _HEADS))],
        out_shape=[jax.ShapeDtypeStruct((_N, _HH), jnp.float32),
                   jax.ShapeDtypeStruct((_N, _HEADS), jnp.float32),
                   jax.ShapeDtypeStruct((_N, _HEADS), jnp.float32)],
    )(x, W, a_s, a_d)


def _proj2(numer, denom, b, W, a_s, a_d):
    grid = _N // _BN
    return pl.pallas_call(
        _proj2_body,
        grid=(grid,),
        in_specs=[_rows((_BN, _HH)), _rows((_BN, _HEADS)), _full((1, _HH)),
                  _full((_HH, _HH)), _full((_HEADS, _HID)), _full((_HEADS, _HID))],
        out_specs=[_rows((_BN, _HH)), _rows((_BN, _HEADS)), _rows((_BN, _HEADS))],
        out_shape=[jax.ShapeDtypeStruct((_N, _HH), jnp.float32),
                   jax.ShapeDtypeStruct((_N, _HEADS), jnp.float32),
                   jax.ShapeDtypeStruct((_N, _HEADS), jnp.float32)],
    )(numer, denom, b.reshape(1, _HH), W, a_s, a_d)


def _pool_fc(numer, denom, b, batch, Wfc, bfc):
    grid = _N // _BN
    return pl.pallas_call(
        _pool_body,
        grid=(grid,),
        in_specs=[_rows((_BN, _HH)), _rows((_BN, _HEADS)), _full((1, _HH)),
                  _rows((_BN, 1)), _full((_HH, _C)), _full((1, _C))],
        out_specs=_full((_G, _C)),
        out_shape=jax.ShapeDtypeStruct((_G, _C), jnp.float32),
        scratch_shapes=[pltpu.VMEM((_G, _HH), jnp.float32),
                        pltpu.VMEM((_G, 1), jnp.float32)],
    )(numer, denom, b.reshape(1, _HH), batch.reshape(_N, 1).astype(jnp.int32),
      Wfc, bfc.reshape(1, _C))


def _edge_aggregate(h, asrc, adst, src, dst):
    al = asrc[src] + adst[dst]
    al = jnp.where(al > 0, al, 0.2 * al)
    e = jnp.exp(al)
    denom = jax.ops.segment_sum(e, dst, num_segments=_N)
    msg = (h[src].reshape(-1, _HEADS, _HID) * e[:, :, None]).reshape(-1, _HH)
    numer = jax.ops.segment_sum(msg, dst, num_segments=_N)
    return numer, denom


def kernel(x, edge_index, batch, W1, a_s1, a_d1, b1, W2, a_s2, a_d2, b2, Wfc, bfc):
    src = edge_index[0]
    dst = edge_index[1]
    h1, asrc1, adst1 = _proj1(x, W1, a_s1, a_d1)
    num1, den1 = _edge_aggregate(h1, asrc1, adst1, src, dst)
    h2, asrc2, adst2 = _proj2(num1, den1, b1, W2, a_s2, a_d2)
    num2, den2 = _edge_aggregate(h2, asrc2, adst2, src, dst)
    return _pool_fc(num2, den2, b2, batch, Wfc, bfc)
